# Initial kernel scaffold; baseline (speedup 1.0000x reference)
import jax
import jax.numpy as jnp
from jax.experimental import pallas as pl


def kernel(x):
    raise NotImplementedError("write your pallas kernel here")


# speedup vs baseline: 3.1418x; 3.1418x over previous
import functools

import jax
import jax.numpy as jnp
from jax import lax
from jax.experimental import pallas as pl

_K = 64
_ROWS_PER_BLOCK = 8


def _tc_body(x_ref, o_ref):
    x = x_ref[...]
    rows, n = x.shape

    bits = lax.bitcast_convert_type(x, jnp.int32)
    ub = lax.bitcast_convert_type(bits, jnp.uint32)
    ukey = jnp.where(bits < 0, ~ub, ub | jnp.uint32(0x80000000))

    def tbit(i, t):
        bit = jnp.uint32(1) << (31 - i).astype(jnp.uint32)
        t_try = t | bit
        cnt = jnp.sum((ukey >= t_try).astype(jnp.int32), axis=1, keepdims=True)
        return jnp.where(cnt >= _K, t_try, t)

    t = lax.fori_loop(0, 32, tbit, jnp.zeros((rows, 1), jnp.uint32))

    gt = ukey > t
    eq = ukey == t
    cnt_gt = jnp.sum(gt.astype(jnp.int32), axis=1, keepdims=True)
    r = _K - cnt_gt

    pos = lax.broadcasted_iota(jnp.int32, (rows, n), 1)
    eqi = eq.astype(jnp.int32)

    def mbit(i, m):
        m_try = m + (jnp.int32(1) << (15 - i))
        ceq = jnp.sum(eqi * (pos < m_try).astype(jnp.int32), axis=1,
                      keepdims=True)
        return jnp.where(ceq <= r, m_try, m)

    m = lax.fori_loop(0, 16, mbit, jnp.zeros((rows, 1), jnp.int32))

    keep = gt | (eq & (pos < m))
    o_ref[...] = jnp.where(keep, x, jnp.float32(0.0))


def kernel(x):
    rows, n = x.shape
    grid = rows // _ROWS_PER_BLOCK
    return pl.pallas_call(
        _tc_body,
        grid=(grid,),
        in_specs=[pl.BlockSpec((_ROWS_PER_BLOCK, n), lambda i: (i, 0))],
        out_specs=pl.BlockSpec((_ROWS_PER_BLOCK, n), lambda i: (i, 0)),
        out_shape=jax.ShapeDtypeStruct((rows, n), x.dtype),
    )(x)
